# baseline (speedup 1.0000x reference)
import functools

import jax
import jax.numpy as jnp
from jax import lax
from jax.experimental import pallas as pl
from jax.experimental.pallas import tpu as pltpu
from jax.experimental.pallas import tpu_sc as plsc

N_QUANTIZER = 4
N_CODEWORD = 8192
LEN_SUBVEC = 256
BATCH = 2048

BT = 1024
NB = BATCH // BT


def _argmin_body(x_ref, cb_ref, idx_ref, csq_ref):
    b = pl.program_id(0)
    cb = cb_ref[0]

    @pl.when(b == 0)
    def _():
        csq_ref[0, :] = jnp.sum(cb * cb, axis=1)

    xs = x_ref[...]
    x_sq = jnp.sum(xs * xs, axis=1, keepdims=True)
    mm2 = lax.dot_general(xs * (-2.0), cb, (((1,), (1,)), ((), ())),
                          preferred_element_type=jnp.float32)

    RG = 128
    W = 128
    nchunk = N_CODEWORD // W
    for rg in range(BT // RG):
        rows = slice(rg * RG, (rg + 1) * RG)
        xsq_rg = x_sq[rows]
        bv = None
        for c in range(nchunk):
            cols = slice(c * W, (c + 1) * W)
            chunk = (xsq_rg + mm2[rows, cols]) + csq_ref[0, cols][None, :]
            if bv is None:
                bv = chunk
                bc = jnp.zeros((RG, W), jnp.int32)
            else:
                m = chunk < bv
                bv = jnp.where(m, chunk, bv)
                bc = jnp.where(m, jnp.int32(c), bc)
        gidx = bc * W + lax.broadcasted_iota(jnp.int32, (RG, W), 1)
        mn = jnp.min(bv, axis=1, keepdims=True)
        idx = jnp.min(jnp.where(bv == mn, gidx, jnp.int32(N_CODEWORD)),
                      axis=1)
        idx_ref[0, 0, pl.ds(b * BT + rg * RG, RG)] = idx


def _tc_argmin(x, codebooks, d):
    return pl.pallas_call(
        _argmin_body,
        grid=(NB,),
        in_specs=[
            pl.BlockSpec((BT, LEN_SUBVEC), lambda b: (b, d)),
            pl.BlockSpec((1, N_CODEWORD, LEN_SUBVEC), lambda b: (d, 0, 0)),
        ],
        out_specs=pl.BlockSpec((1, 1, BATCH), lambda b: (0, 0, 0)),
        out_shape=jax.ShapeDtypeStruct((1, 1, BATCH), jnp.int32),
        scratch_shapes=[pltpu.VMEM((1, N_CODEWORD), jnp.float32)],
    )(x, codebooks)


@functools.lru_cache(maxsize=1)
def _make_sc_gather():
    info = plsc.get_sparse_core_info()
    nc, ns = info.num_cores, info.num_subcores
    nw = nc * ns
    rpw = BATCH // nw
    mesh = plsc.VectorSubcoreMesh(core_axis_name="c", subcore_axis_name="s")

    @functools.partial(
        pl.kernel, mesh=mesh,
        out_type=jax.ShapeDtypeStruct((nw, rpw, LEN_SUBVEC), jnp.float32),
        scratch_types=[
            pltpu.VMEM((1, rpw), jnp.int32),
            pltpu.VMEM((rpw, LEN_SUBVEC), jnp.float32),
            pltpu.SemaphoreType.DMA,
        ],
    )
    def gather(table_hbm, gidx_hbm, out_hbm, gidx_v, rows_v, sem):
        wid = lax.axis_index("s") * nc + lax.axis_index("c")
        pltpu.sync_copy(gidx_hbm.at[pl.ds(wid, 1)], gidx_v)
        pltpu.async_copy(table_hbm.at[gidx_v.at[0]], rows_v, sem).wait()
        pltpu.sync_copy(rows_v, out_hbm.at[wid])

    return gather


def kernel(x, codebooks):
    table = codebooks.reshape(N_QUANTIZER * N_CODEWORD, LEN_SUBVEC)
    sc_gather = _make_sc_gather()
    nw = 32
    ids, qxs = [], []
    for d in range(N_QUANTIZER):
        id_d = _tc_argmin(x, codebooks, d).reshape(BATCH)
        gidx = (id_d + jnp.int32(d * N_CODEWORD)).reshape(nw, BATCH // nw)
        qx_d = sc_gather(table, gidx)
        ids.append(id_d)
        qxs.append(qx_d.reshape(BATCH, LEN_SUBVEC))
    q_x = jnp.stack(qxs, axis=1).reshape(BATCH, N_QUANTIZER * LEN_SUBVEC)
    id_x = jnp.stack(ids, axis=0)
    return (q_x, id_x)
